# baseline (speedup 1.0000x reference)
import functools

import jax
import jax.numpy as jnp
from jax import lax
from jax.experimental import pallas as pl
from jax.experimental.pallas import tpu as pltpu
from jax.experimental.pallas import tpu_sc as plsc

N_USER = 5000
N_ITEM = 5000
N = N_USER + N_ITEM
E_HALF = 160000
D = 128
NC = 2
NS = 16
CH = 128
E_SUB = E_HALF // NS
NBUF = 4
NCHUNK = NBUF * (-(-E_SUB // (CH * NBUF)))
E_SUB_PAD = NCHUNK * CH
E_PAD = NS * E_SUB_PAD
ACC_ROWS = 5120
DUMP = 5000
ZROWS = 320


def _spmm_sc(width):
  mesh = plsc.VectorSubcoreMesh(core_axis_name="c", subcore_axis_name="s")

  @functools.partial(
      pl.kernel,
      out_type=jax.ShapeDtypeStruct((N, width), jnp.float32),
      mesh=mesh,
      compiler_params=pltpu.CompilerParams(use_tc_tiling_on_sc=False),
      scratch_types=[
          pltpu.VMEM_SHARED((ACC_ROWS, width), jnp.float32),
          pltpu.VMEM((E_SUB_PAD,), jnp.int32),
          pltpu.VMEM((NCHUNK, CH), jnp.int32),
          [pltpu.VMEM((CH, width), jnp.float32)] * NBUF,
          [pltpu.SemaphoreType.DMA] * NBUF,
          [pltpu.SemaphoreType.DMA] * NBUF,
      ],
  )
  def spmm(table_hbm, src_hbm, dst_hbm, zeros_hbm, out_hbm,
           acc, sidx, didx, rows, gsems, ssems):
    c = lax.axis_index("c")
    s = lax.axis_index("s")
    pltpu.sync_copy(zeros_hbm, acc.at[pl.ds(s * ZROWS, ZROWS)])

    w = c * NS + s
    pltpu.sync_copy(src_hbm.at[pl.ds(w * E_SUB_PAD, E_SUB_PAD)], sidx)
    pltpu.sync_copy(dst_hbm.at[pl.ds(w * NCHUNK, NCHUNK)], didx)
    plsc.subcore_barrier()

    def gather(k, b):
      pltpu.async_copy(table_hbm.at[sidx.at[pl.ds(k * CH, CH)]],
                       rows[b], gsems[b])

    def wait_gather(b):
      pltpu.make_async_copy(table_hbm.at[sidx.at[pl.ds(0, CH)]],
                            rows[b], gsems[b]).wait()

    def wait_scatter(b):
      pltpu.make_async_copy(rows[b], acc.at[didx.at[0]], ssems[b]).wait()

    for b in range(NBUF):
      gather(b, b)

    def outer(g, _):
      for b in range(NBUF):
        k = g * NBUF + b
        wait_gather(b)
        pltpu.async_copy(rows[b], acc.at[didx.at[k]], ssems[b], add=True)

        @pl.when(k + NBUF < NCHUNK)
        def _():
          wait_scatter(b)
          gather(k + NBUF, b)
      return _

    lax.fori_loop(0, NCHUNK // NBUF, outer, None)
    for b in range(NBUF):
      wait_scatter(b)
    plsc.subcore_barrier()

    out_base = c * N_USER
    pltpu.sync_copy(acc.at[pl.ds(s * 312, 312)],
                    out_hbm.at[pl.ds(out_base + s * 312, 312)])

    @pl.when(s == NS - 1)
    def _():
      pltpu.sync_copy(acc.at[pl.ds(4992, 8)],
                      out_hbm.at[pl.ds(out_base + 4992, 8)])

  return spmm


_spmm16 = _spmm_sc(16)
_spmm128 = _spmm_sc(D)


_GB = 1024 // (NC * NS)


def _batch_gather(ae_hbm, u_hbm, p_hbm, n_hbm,
                  uo_hbm, po_hbm, no_hbm, idx, rows, sem):
  c = lax.axis_index("c")
  s = lax.axis_index("s")
  w = s * NC + c
  base = w * _GB
  for src_idx, out in ((u_hbm, uo_hbm), (p_hbm, po_hbm), (n_hbm, no_hbm)):
    pltpu.sync_copy(src_idx.at[pl.ds(base, _GB)], idx)
    pltpu.async_copy(ae_hbm.at[idx], rows, sem).wait()
    pltpu.sync_copy(rows, out.at[pl.ds(base, _GB)])


_gather_call = functools.partial(
    pl.kernel,
    out_type=(jax.ShapeDtypeStruct((1024, 4 * D), jnp.float32),) * 3,
    mesh=plsc.VectorSubcoreMesh(core_axis_name="c", subcore_axis_name="s"),
    scratch_types=[
        pltpu.VMEM((_GB,), jnp.int32),
        pltpu.VMEM((_GB, 4 * D), jnp.float32),
        pltpu.SemaphoreType.DMA,
    ],
)(_batch_gather)


_BLK = 1000


def _dense_body(emb_ref, g_ref, cp1_ref, rd_ref, wg_ref, we_ref,
                out_ref, s_out_ref):
  e = emb_ref[...]
  rd = rd_ref[...]
  s_in = e * rd
  a = e * cp1_ref[...]
  b = s_in * g_ref[...]
  res = lax.dot_general(a, wg_ref[...], (((1,), (1,)), ((), ())),
                        preferred_element_type=jnp.float32)
  res += lax.dot_general(b, we_ref[...], (((1,), (1,)), ((), ())),
                         preferred_element_type=jnp.float32)
  res = jnp.where(res >= 0, res, 0.2 * res)
  nrm = jnp.maximum(jnp.sqrt(jnp.sum(res * res, axis=1, keepdims=True)), 1e-12)
  o = res / nrm
  out_ref[...] = o
  s_out_ref[...] = o * rd


def _dense_layer(emb, g, cp1, rd, Wg, We):
  row_spec = pl.BlockSpec((_BLK, D), lambda i: (i, 0))
  col_spec = pl.BlockSpec((_BLK, 1), lambda i: (i, 0))
  w_spec = pl.BlockSpec((D, D), lambda i: (0, 0))
  return pl.pallas_call(
      _dense_body,
      grid=(N // _BLK,),
      in_specs=[row_spec, row_spec, col_spec, col_spec, w_spec, w_spec],
      out_specs=(row_spec, row_spec),
      out_shape=(jax.ShapeDtypeStruct((N, D), jnp.float32),) * 2,
  )(emb, g, cp1, rd, Wg, We)


def _pad_edges(x, padval):
  x = x.reshape(NS, E_SUB)
  return jnp.pad(x, ((0, 0), (0, E_SUB_PAD - E_SUB)),
                 constant_values=padval).reshape(-1)


def kernel(user, pos_item, neg_item, edge_index, feature,
           W_gcn_0, W_enh_0, W_gcn_1, W_enh_1, W_gcn_2, W_enh_2):
  edge = edge_index.astype(jnp.int32)
  src_flat = jnp.concatenate([
      _pad_edges(edge[0, E_HALF:], 0),
      _pad_edges(edge[0, :E_HALF], 0)])
  dst_flat = jnp.concatenate([
      _pad_edges(edge[1, E_HALF:], DUMP),
      _pad_edges(edge[1, :E_HALF] - N_USER, DUMP)]).reshape(-1, CH)

  z16 = jnp.zeros((ZROWS, 16), jnp.float32)
  z128 = jnp.zeros((ZROWS, D), jnp.float32)
  ones16 = jnp.ones((N, 16), jnp.float32)

  deg16 = _spmm16(ones16, src_flat, dst_flat, z16)
  rd16 = lax.rsqrt(jnp.maximum(deg16, 1.0))
  c16 = _spmm16(rd16, src_flat, dst_flat, z16)

  rd = rd16[:, :1]
  cp1 = 1.0 + c16[:, :1] * rd

  emb = feature
  s = feature * rd
  all_e = [feature]
  for Wg, We in ((W_gcn_0, W_enh_0), (W_gcn_1, W_enh_1), (W_gcn_2, W_enh_2)):
    g = _spmm128(s, src_flat, dst_flat, z128)
    emb, s = _dense_layer(emb, g, cp1, rd, Wg, We)
    all_e.append(emb)

  all_e = jnp.concatenate(all_e, axis=1)
  return _gather_call(all_e, user.astype(jnp.int32),
                      pos_item.astype(jnp.int32) + N_USER,
                      neg_item.astype(jnp.int32) + N_USER)


# speedup vs baseline: 1.0415x; 1.0415x over previous
import functools

import jax
import jax.numpy as jnp
from jax import lax
from jax.experimental import pallas as pl
from jax.experimental.pallas import tpu as pltpu
from jax.experimental.pallas import tpu_sc as plsc

N_USER = 5000
N_ITEM = 5000
N = N_USER + N_ITEM
E_HALF = 160000
D = 128
NC = 2
NS = 16
CH = 128
E_SUB = E_HALF // NS
NBUF = 4
NCHUNK = NBUF * (-(-E_SUB // (CH * NBUF)))
E_SUB_PAD = NCHUNK * CH
E_PAD = NS * E_SUB_PAD
ACC_ROWS = 5120
DUMP = 5000
ZROWS = 320


def _spmm_sc(width):
  mesh = plsc.VectorSubcoreMesh(core_axis_name="c", subcore_axis_name="s")

  @functools.partial(
      pl.kernel,
      out_type=jax.ShapeDtypeStruct((N, width), jnp.float32),
      mesh=mesh,
      compiler_params=pltpu.CompilerParams(use_tc_tiling_on_sc=False),
      scratch_types=[
          pltpu.VMEM_SHARED((ACC_ROWS, width), jnp.float32),
          pltpu.VMEM((E_SUB_PAD,), jnp.int32),
          pltpu.VMEM((NCHUNK, CH), jnp.int32),
          [pltpu.VMEM((CH, width), jnp.float32)] * NBUF,
          [pltpu.SemaphoreType.DMA] * NBUF,
          [pltpu.SemaphoreType.DMA] * NBUF,
      ],
  )
  def spmm(table_hbm, src_hbm, dst_hbm, zeros_hbm, out_hbm,
           acc, sidx, didx, rows, gsems, ssems):
    c = lax.axis_index("c")
    s = lax.axis_index("s")
    pltpu.sync_copy(zeros_hbm, acc.at[pl.ds(s * ZROWS, ZROWS)])

    w = c * NS + s
    pltpu.sync_copy(src_hbm.at[pl.ds(w * E_SUB_PAD, E_SUB_PAD)], sidx)
    pltpu.sync_copy(dst_hbm.at[pl.ds(w * NCHUNK, NCHUNK)], didx)
    plsc.subcore_barrier()

    def gather(k, b):
      pltpu.async_copy(table_hbm.at[sidx.at[pl.ds(k * CH, CH)]],
                       rows[b], gsems[b])

    def wait_gather(b):
      pltpu.make_async_copy(table_hbm.at[sidx.at[pl.ds(0, CH)]],
                            rows[b], gsems[b]).wait()

    def wait_scatter(b):
      pltpu.make_async_copy(rows[b], acc.at[didx.at[0]], ssems[b]).wait()

    for b in range(NBUF):
      gather(b, b)

    def outer(g, _):
      for b in range(NBUF):
        k = g * NBUF + b
        wait_gather(b)
        pltpu.async_copy(rows[b], acc.at[didx.at[k]], ssems[b], add=True)

        @pl.when(k + NBUF < NCHUNK)
        def _():
          wait_scatter(b)
          gather(k + NBUF, b)
      return _

    lax.fori_loop(0, NCHUNK // NBUF, outer, None)
    for b in range(NBUF):
      wait_scatter(b)
    plsc.subcore_barrier()

    out_base = c * N_USER
    pltpu.sync_copy(acc.at[pl.ds(s * 312, 312)],
                    out_hbm.at[pl.ds(out_base + s * 312, 312)])

    @pl.when(s == NS - 1)
    def _():
      pltpu.sync_copy(acc.at[pl.ds(4992, 8)],
                      out_hbm.at[pl.ds(out_base + 4992, 8)])

  return spmm


_spmm16 = _spmm_sc(16)
_spmm128 = _spmm_sc(D)


@functools.partial(
    pl.kernel,
    out_type=jax.ShapeDtypeStruct((N, 16), jnp.float32),
    mesh=plsc.VectorSubcoreMesh(core_axis_name="c", subcore_axis_name="s"),
    compiler_params=pltpu.CompilerParams(use_tc_tiling_on_sc=False),
    scratch_types=[
        pltpu.VMEM_SHARED((ACC_ROWS, 16), jnp.float32),
        pltpu.VMEM((NCHUNK, CH), jnp.int32),
        pltpu.VMEM((CH, 16), jnp.float32),
        [pltpu.SemaphoreType.DMA] * NBUF,
    ],
)
def _degree(dst_hbm, zeros_hbm, out_hbm, acc, didx, ones, ssems):
  c = lax.axis_index("c")
  s = lax.axis_index("s")
  pltpu.sync_copy(zeros_hbm, acc.at[pl.ds(s * ZROWS, ZROWS)])
  w = c * NS + s
  pltpu.sync_copy(dst_hbm.at[pl.ds(w * NCHUNK, NCHUNK)], didx)
  for i in range(CH):
    ones[i] = jnp.ones((16,), jnp.float32)
  plsc.subcore_barrier()

  def wait_scatter(b):
    pltpu.make_async_copy(ones, acc.at[didx.at[0]], ssems[b]).wait()

  def outer(g, _):
    for b in range(NBUF):
      @pl.when(g > 0)
      def _():
        wait_scatter(b)
      pltpu.async_copy(ones, acc.at[didx.at[g * NBUF + b]], ssems[b],
                       add=True)
    return _

  lax.fori_loop(0, NCHUNK // NBUF, outer, None)
  for b in range(NBUF):
    wait_scatter(b)
  plsc.subcore_barrier()

  out_base = c * N_USER
  pltpu.sync_copy(acc.at[pl.ds(s * 312, 312)],
                  out_hbm.at[pl.ds(out_base + s * 312, 312)])

  @pl.when(s == NS - 1)
  def _():
    pltpu.sync_copy(acc.at[pl.ds(4992, 8)],
                    out_hbm.at[pl.ds(out_base + 4992, 8)])


_GB = 1024 // (NC * NS)


def _batch_gather(t0_hbm, t1_hbm, t2_hbm, t3_hbm, u_hbm, p_hbm, n_hbm,
                  uo_hbm, po_hbm, no_hbm, idx, rows, sem):
  c = lax.axis_index("c")
  s = lax.axis_index("s")
  w = s * NC + c
  base = w * _GB
  for src_idx, out in ((u_hbm, uo_hbm), (p_hbm, po_hbm), (n_hbm, no_hbm)):
    pltpu.sync_copy(src_idx.at[pl.ds(base, _GB)], idx)
    for t, table in enumerate((t0_hbm, t1_hbm, t2_hbm, t3_hbm)):
      pltpu.async_copy(table.at[idx], rows, sem).wait()
      pltpu.sync_copy(rows, out.at[pl.ds(base, _GB), pl.ds(t * D, D)])


_gather_call = functools.partial(
    pl.kernel,
    out_type=(jax.ShapeDtypeStruct((1024, 4 * D), jnp.float32),) * 3,
    mesh=plsc.VectorSubcoreMesh(core_axis_name="c", subcore_axis_name="s"),
    scratch_types=[
        pltpu.VMEM((_GB,), jnp.int32),
        pltpu.VMEM((_GB, D), jnp.float32),
        pltpu.SemaphoreType.DMA,
    ],
)(_batch_gather)


_BLK = 1000


def _dense_body(emb_ref, g_ref, cp1_ref, rd_ref, wg_ref, we_ref,
                out_ref, s_out_ref):
  e = emb_ref[...]
  rd = rd_ref[...]
  s_in = e * rd
  a = e * cp1_ref[...]
  b = s_in * g_ref[...]
  res = lax.dot_general(a, wg_ref[...], (((1,), (1,)), ((), ())),
                        preferred_element_type=jnp.float32)
  res += lax.dot_general(b, we_ref[...], (((1,), (1,)), ((), ())),
                         preferred_element_type=jnp.float32)
  res = jnp.where(res >= 0, res, 0.2 * res)
  nrm = jnp.maximum(jnp.sqrt(jnp.sum(res * res, axis=1, keepdims=True)), 1e-12)
  o = res / nrm
  out_ref[...] = o
  s_out_ref[...] = o * rd


def _dense_layer(emb, g, cp1, rd, Wg, We):
  row_spec = pl.BlockSpec((_BLK, D), lambda i: (i, 0))
  col_spec = pl.BlockSpec((_BLK, 1), lambda i: (i, 0))
  w_spec = pl.BlockSpec((D, D), lambda i: (0, 0))
  return pl.pallas_call(
      _dense_body,
      grid=(N // _BLK,),
      in_specs=[row_spec, row_spec, col_spec, col_spec, w_spec, w_spec],
      out_specs=(row_spec, row_spec),
      out_shape=(jax.ShapeDtypeStruct((N, D), jnp.float32),) * 2,
  )(emb, g, cp1, rd, Wg, We)


def _pad_edges(x, padval):
  x = x.reshape(NS, E_SUB)
  return jnp.pad(x, ((0, 0), (0, E_SUB_PAD - E_SUB)),
                 constant_values=padval).reshape(-1)


def kernel(user, pos_item, neg_item, edge_index, feature,
           W_gcn_0, W_enh_0, W_gcn_1, W_enh_1, W_gcn_2, W_enh_2):
  edge = edge_index.astype(jnp.int32)
  src_flat = jnp.concatenate([
      _pad_edges(edge[0, E_HALF:], 0),
      _pad_edges(edge[0, :E_HALF], 0)])
  dst_flat = jnp.concatenate([
      _pad_edges(edge[1, E_HALF:], DUMP),
      _pad_edges(edge[1, :E_HALF] - N_USER, DUMP)]).reshape(-1, CH)

  z16 = jnp.zeros((ZROWS, 16), jnp.float32)
  z128 = jnp.zeros((ZROWS, D), jnp.float32)

  deg16 = _degree(dst_flat, z16)
  rd16 = lax.rsqrt(jnp.maximum(deg16, 1.0))
  c16 = _spmm16(rd16, src_flat, dst_flat, z16)

  rd = rd16[:, :1]
  cp1 = 1.0 + c16[:, :1] * rd

  emb = feature
  s = feature * rd
  all_e = [feature]
  for Wg, We in ((W_gcn_0, W_enh_0), (W_gcn_1, W_enh_1), (W_gcn_2, W_enh_2)):
    g = _spmm128(s, src_flat, dst_flat, z128)
    emb, s = _dense_layer(emb, g, cp1, rd, Wg, We)
    all_e.append(emb)

  return _gather_call(*all_e, user.astype(jnp.int32),
                      pos_item.astype(jnp.int32) + N_USER,
                      neg_item.astype(jnp.int32) + N_USER)


# speedup vs baseline: 1.0581x; 1.0159x over previous
import functools

import jax
import jax.numpy as jnp
from jax import lax
from jax.experimental import pallas as pl
from jax.experimental.pallas import tpu as pltpu
from jax.experimental.pallas import tpu_sc as plsc

N_USER = 5000
N_ITEM = 5000
N = N_USER + N_ITEM
E_HALF = 160000
D = 128
NC = 2
NS = 16
CH = 128
E_SUB = E_HALF // NS
NBUF = 4
NCHUNK = NBUF * (-(-E_SUB // (CH * NBUF)))
E_SUB_PAD = NCHUNK * CH
E_PAD = NS * E_SUB_PAD
ACC_ROWS = 5120
DUMP = 5000
ZROWS = 320


def _spmm_sc(width, nbuf=NBUF):
  mesh = plsc.VectorSubcoreMesh(core_axis_name="c", subcore_axis_name="s")

  @functools.partial(
      pl.kernel,
      out_type=jax.ShapeDtypeStruct((N, width), jnp.float32),
      mesh=mesh,
      compiler_params=pltpu.CompilerParams(use_tc_tiling_on_sc=False),
      scratch_types=[
          pltpu.VMEM_SHARED((ACC_ROWS, width), jnp.float32),
          pltpu.VMEM((E_SUB_PAD,), jnp.int32),
          pltpu.VMEM((NCHUNK, CH), jnp.int32),
          [pltpu.VMEM((CH, width), jnp.float32)] * nbuf,
          [pltpu.SemaphoreType.DMA] * nbuf,
          [pltpu.SemaphoreType.DMA] * nbuf,
      ],
  )
  def spmm(table_hbm, src_hbm, dst_hbm, zeros_hbm, out_hbm,
           acc, sidx, didx, rows, gsems, ssems):
    c = lax.axis_index("c")
    s = lax.axis_index("s")
    pltpu.sync_copy(zeros_hbm, acc.at[pl.ds(s * ZROWS, ZROWS)])

    w = c * NS + s
    pltpu.sync_copy(src_hbm.at[pl.ds(w * E_SUB_PAD, E_SUB_PAD)], sidx)
    pltpu.sync_copy(dst_hbm.at[pl.ds(w * NCHUNK, NCHUNK)], didx)
    plsc.subcore_barrier()

    def gather(k, b):
      pltpu.async_copy(table_hbm.at[sidx.at[pl.ds(k * CH, CH)]],
                       rows[b], gsems[b])

    def wait_gather(b):
      pltpu.make_async_copy(table_hbm.at[sidx.at[pl.ds(0, CH)]],
                            rows[b], gsems[b]).wait()

    def wait_scatter(b):
      pltpu.make_async_copy(rows[b], acc.at[didx.at[0]], ssems[b]).wait()

    for b in range(nbuf):
      gather(b, b)

    def outer(g, _):
      for b in range(nbuf):
        k = g * nbuf + b
        wait_gather(b)
        pltpu.async_copy(rows[b], acc.at[didx.at[k]], ssems[b], add=True)

        @pl.when(k + nbuf < NCHUNK)
        def _():
          wait_scatter(b)
          gather(k + nbuf, b)
      return _

    lax.fori_loop(0, NCHUNK // nbuf, outer, None)
    for b in range(nbuf):
      wait_scatter(b)
    plsc.subcore_barrier()

    out_base = c * N_USER
    pltpu.sync_copy(acc.at[pl.ds(s * 312, 312)],
                    out_hbm.at[pl.ds(out_base + s * 312, 312)])

    @pl.when(s == NS - 1)
    def _():
      pltpu.sync_copy(acc.at[pl.ds(4992, 8)],
                      out_hbm.at[pl.ds(out_base + 4992, 8)])

  return spmm


_spmm128 = _spmm_sc(D)
_spmm144 = _spmm_sc(D + 16, nbuf=2)


@functools.partial(
    pl.kernel,
    out_type=jax.ShapeDtypeStruct((N, 16), jnp.float32),
    mesh=plsc.VectorSubcoreMesh(core_axis_name="c", subcore_axis_name="s"),
    compiler_params=pltpu.CompilerParams(use_tc_tiling_on_sc=False),
    scratch_types=[
        pltpu.VMEM_SHARED((ACC_ROWS, 16), jnp.float32),
        pltpu.VMEM((NCHUNK, CH), jnp.int32),
        pltpu.VMEM((CH, 16), jnp.float32),
        [pltpu.SemaphoreType.DMA] * NBUF,
    ],
)
def _degree(dst_hbm, zeros_hbm, out_hbm, acc, didx, ones, ssems):
  c = lax.axis_index("c")
  s = lax.axis_index("s")
  pltpu.sync_copy(zeros_hbm, acc.at[pl.ds(s * ZROWS, ZROWS)])
  w = c * NS + s
  pltpu.sync_copy(dst_hbm.at[pl.ds(w * NCHUNK, NCHUNK)], didx)
  for i in range(CH):
    ones[i] = jnp.ones((16,), jnp.float32)
  plsc.subcore_barrier()

  def wait_scatter(b):
    pltpu.make_async_copy(ones, acc.at[didx.at[0]], ssems[b]).wait()

  def outer(g, _):
    for b in range(NBUF):
      @pl.when(g > 0)
      def _():
        wait_scatter(b)
      pltpu.async_copy(ones, acc.at[didx.at[g * NBUF + b]], ssems[b],
                       add=True)
    return _

  lax.fori_loop(0, NCHUNK // NBUF, outer, None)
  for b in range(NBUF):
    wait_scatter(b)
  plsc.subcore_barrier()

  out_base = c * N_USER
  pltpu.sync_copy(acc.at[pl.ds(s * 312, 312)],
                  out_hbm.at[pl.ds(out_base + s * 312, 312)])

  @pl.when(s == NS - 1)
  def _():
    pltpu.sync_copy(acc.at[pl.ds(4992, 8)],
                    out_hbm.at[pl.ds(out_base + 4992, 8)])


_GB = 1024 // (NC * NS)


def _batch_gather(t0_hbm, t1_hbm, t2_hbm, t3_hbm, u_hbm, p_hbm, n_hbm,
                  uo_hbm, po_hbm, no_hbm, idx, rows, sem):
  c = lax.axis_index("c")
  s = lax.axis_index("s")
  w = s * NC + c
  base = w * _GB
  for src_idx, out in ((u_hbm, uo_hbm), (p_hbm, po_hbm), (n_hbm, no_hbm)):
    pltpu.sync_copy(src_idx.at[pl.ds(base, _GB)], idx)
    for t, table in enumerate((t0_hbm, t1_hbm, t2_hbm, t3_hbm)):
      pltpu.async_copy(table.at[idx], rows, sem).wait()
      pltpu.sync_copy(rows, out.at[pl.ds(base, _GB), pl.ds(t * D, D)])


_gather_call = functools.partial(
    pl.kernel,
    out_type=(jax.ShapeDtypeStruct((1024, 4 * D), jnp.float32),) * 3,
    mesh=plsc.VectorSubcoreMesh(core_axis_name="c", subcore_axis_name="s"),
    scratch_types=[
        pltpu.VMEM((_GB,), jnp.int32),
        pltpu.VMEM((_GB, D), jnp.float32),
        pltpu.SemaphoreType.DMA,
    ],
)(_batch_gather)


_BLK = 1000


def _dense_body(emb_ref, g_ref, cp1_ref, rd_ref, wg_ref, we_ref,
                out_ref, s_out_ref):
  e = emb_ref[...]
  rd = rd_ref[...]
  s_in = e * rd
  a = e * cp1_ref[...]
  b = s_in * g_ref[...]
  res = lax.dot_general(a, wg_ref[...], (((1,), (1,)), ((), ())),
                        preferred_element_type=jnp.float32)
  res += lax.dot_general(b, we_ref[...], (((1,), (1,)), ((), ())),
                         preferred_element_type=jnp.float32)
  res = jnp.where(res >= 0, res, 0.2 * res)
  nrm = jnp.maximum(jnp.sqrt(jnp.sum(res * res, axis=1, keepdims=True)), 1e-12)
  o = res / nrm
  out_ref[...] = o
  s_out_ref[...] = o * rd


def _dense_layer(emb, g, cp1, rd, Wg, We):
  row_spec = pl.BlockSpec((_BLK, D), lambda i: (i, 0))
  col_spec = pl.BlockSpec((_BLK, 1), lambda i: (i, 0))
  w_spec = pl.BlockSpec((D, D), lambda i: (0, 0))
  return pl.pallas_call(
      _dense_body,
      grid=(N // _BLK,),
      in_specs=[row_spec, row_spec, col_spec, col_spec, w_spec, w_spec],
      out_specs=(row_spec, row_spec),
      out_shape=(jax.ShapeDtypeStruct((N, D), jnp.float32),) * 2,
  )(emb, g, cp1, rd, Wg, We)


def _pad_edges(x, padval):
  x = x.reshape(NS, E_SUB)
  return jnp.pad(x, ((0, 0), (0, E_SUB_PAD - E_SUB)),
                 constant_values=padval).reshape(-1)


def kernel(user, pos_item, neg_item, edge_index, feature,
           W_gcn_0, W_enh_0, W_gcn_1, W_enh_1, W_gcn_2, W_enh_2):
  edge = edge_index.astype(jnp.int32)
  src_flat = jnp.concatenate([
      _pad_edges(edge[0, E_HALF:], 0),
      _pad_edges(edge[0, :E_HALF], 0)])
  dst_flat = jnp.concatenate([
      _pad_edges(edge[1, E_HALF:], DUMP),
      _pad_edges(edge[1, :E_HALF] - N_USER, DUMP)]).reshape(-1, CH)

  z16 = jnp.zeros((ZROWS, 16), jnp.float32)
  z128 = jnp.zeros((ZROWS, D), jnp.float32)
  z144 = jnp.zeros((ZROWS, D + 16), jnp.float32)

  deg16 = _degree(dst_flat, z16)
  rd16 = lax.rsqrt(jnp.maximum(deg16, 1.0))
  rd = rd16[:, :1]

  emb = feature
  s = feature * rd
  all_e = [feature]
  cp1 = None
  for layer, (Wg, We) in enumerate(((W_gcn_0, W_enh_0), (W_gcn_1, W_enh_1),
                                    (W_gcn_2, W_enh_2))):
    if layer == 0:
      g144 = _spmm144(jnp.concatenate([s, rd16], axis=1),
                      src_flat, dst_flat, z144)
      g = g144[:, :D]
      cp1 = 1.0 + g144[:, D:D + 1] * rd
    else:
      g = _spmm128(s, src_flat, dst_flat, z128)
    emb, s = _dense_layer(emb, g, cp1, rd, Wg, We)
    all_e.append(emb)

  return _gather_call(*all_e, user.astype(jnp.int32),
                      pos_item.astype(jnp.int32) + N_USER,
                      neg_item.astype(jnp.int32) + N_USER)
